# baseline (speedup 1.0000x reference)
import functools

import jax
import jax.numpy as jnp
from jax import lax
from jax.experimental import pallas as pl
from jax.experimental.pallas import tpu as pltpu
from jax.experimental.pallas import tpu_sc as plsc

VOCAB = 100000
EMBED_DIM = 128
B_ROWS = 4096
B_COLS = 50
B = B_ROWS * B_COLS

NC = 2
NS = 16
NW = NC * NS
B_PER_W = B // NW
CHUNK = 400
N_CHUNKS = B_PER_W // CHUNK


def _make_kernel():
  mesh = plsc.VectorSubcoreMesh(
      core_axis_name="c", subcore_axis_name="s",
      num_cores=NC, num_subcores=NS)

  @functools.partial(
      pl.kernel,
      out_type=jax.ShapeDtypeStruct((B, EMBED_DIM), jnp.float32),
      mesh=mesh,
      scratch_types=[
          pltpu.VMEM((B_PER_W,), jnp.int32),
          pltpu.VMEM((CHUNK, EMBED_DIM), jnp.float32),
          pltpu.VMEM((CHUNK, EMBED_DIM), jnp.float32),
          pltpu.SemaphoreType.DMA,
          pltpu.SemaphoreType.DMA,
          pltpu.SemaphoreType.DMA,
          pltpu.SemaphoreType.DMA,
      ],
      compiler_params=pltpu.CompilerParams(use_tc_tiling_on_sc=True),
  )
  def gather_kernel(idx_hbm, table_hbm, out_hbm, idx_v, rows0, rows1,
                    g0, g1, s0, s1):
    wid = lax.axis_index("s") * NC + lax.axis_index("c")
    base = wid * B_PER_W
    pltpu.sync_copy(idx_hbm.at[pl.ds(base, B_PER_W)], idx_v)

    def gather(c, buf, sem):
      return pltpu.make_async_copy(
          table_hbm.at[idx_v.at[pl.ds(c * CHUNK, CHUNK)]], buf, sem)

    def put(c, buf, sem):
      return pltpu.make_async_copy(
          buf, out_hbm.at[pl.ds(base + c * CHUNK, CHUNK)], sem)

    gather(0, rows0, g0).start()
    gather(1, rows1, g1).start()

    def body(i, carry):
      c = i * 2
      gather(c, rows0, g0).wait()
      put(c, rows0, s0).start()
      gather(c + 1, rows1, g1).wait()
      put(c + 1, rows1, s1).start()
      put(c, rows0, s0).wait()

      @pl.when(c + 2 < N_CHUNKS)
      def _():
        gather(c + 2, rows0, g0).start()

      put(c + 1, rows1, s1).wait()

      @pl.when(c + 3 < N_CHUNKS)
      def _():
        gather(c + 3, rows1, g1).start()

      return carry

    lax.fori_loop(0, N_CHUNKS // 2, body, 0)

  return gather_kernel


_GATHER = _make_kernel()


def kernel(x, table):
  idx = x.reshape(-1).astype(jnp.int32)
  out = _GATHER(idx, table)
  return out.reshape(B_ROWS, B_COLS, EMBED_DIM)


# speedup vs baseline: 1.6851x; 1.6851x over previous
import functools

import jax
import jax.numpy as jnp
from jax import lax
from jax.experimental import pallas as pl
from jax.experimental.pallas import tpu as pltpu
from jax.experimental.pallas import tpu_sc as plsc

VOCAB = 100000
EMBED_DIM = 128
B_ROWS = 4096
B_COLS = 50
B = B_ROWS * B_COLS

NC = 2
NS = 16
NW = NC * NS
B_PER_W = B // NW
CHUNK = 400
N_CHUNKS = B_PER_W // CHUNK


def _make_kernel():
  mesh = plsc.VectorSubcoreMesh(
      core_axis_name="c", subcore_axis_name="s",
      num_cores=NC, num_subcores=NS)

  @functools.partial(
      pl.kernel,
      out_type=jax.ShapeDtypeStruct((B_ROWS, B_COLS, EMBED_DIM), jnp.float32),
      mesh=mesh,
      scratch_types=[
          pltpu.VMEM((B_PER_W,), jnp.int32),
          pltpu.VMEM((CHUNK, EMBED_DIM), jnp.float32),
          pltpu.SemaphoreType.DMA,
      ],
      compiler_params=pltpu.CompilerParams(use_tc_tiling_on_sc=True),
  )
  def gather_kernel(idx_hbm, table_hbm, out_hbm, idx_v, rows, gsem):
    wid = lax.axis_index("s") * NC + lax.axis_index("c")
    base = wid * B_PER_W
    maj0 = wid * (B_ROWS // NW)
    pltpu.sync_copy(idx_hbm.at[pl.ds(base, B_PER_W)], idx_v)

    def body(c, carry):
      pltpu.async_copy(
          table_hbm.at[idx_v.at[pl.ds(c * CHUNK, CHUNK)]], rows, gsem).wait()
      for m in range(CHUNK // B_COLS):
        pltpu.sync_copy(rows.at[pl.ds(m * B_COLS, B_COLS)],
                        out_hbm.at[maj0 + c * (CHUNK // B_COLS) + m])
      return carry

    lax.fori_loop(0, N_CHUNKS, body, 0)

  return gather_kernel


_GATHER = _make_kernel()


def kernel(x, table):
  idx = x.reshape(-1).astype(jnp.int32)
  return _GATHER(idx, table)


# speedup vs baseline: 1.6934x; 1.0049x over previous
import functools

import jax
import jax.numpy as jnp
from jax import lax
from jax.experimental import pallas as pl
from jax.experimental.pallas import tpu as pltpu
from jax.experimental.pallas import tpu_sc as plsc

VOCAB = 100000
EMBED_DIM = 128
B_ROWS = 4096
B_COLS = 50
B = B_ROWS * B_COLS

NC = 2
NS = 16
NW = NC * NS
B_PER_W = B // NW
CHUNK = 400
N_CHUNKS = B_PER_W // CHUNK


def _make_kernel():
  mesh = plsc.VectorSubcoreMesh(
      core_axis_name="c", subcore_axis_name="s",
      num_cores=NC, num_subcores=NS)

  @functools.partial(
      pl.kernel,
      out_type=jax.ShapeDtypeStruct((B_ROWS, B_COLS, EMBED_DIM), jnp.float32),
      mesh=mesh,
      scratch_types=[
          pltpu.VMEM((B_PER_W,), jnp.int32),
          pltpu.VMEM((CHUNK, EMBED_DIM), jnp.float32),
          pltpu.SemaphoreType.DMA,
      ],
      compiler_params=pltpu.CompilerParams(
          use_tc_tiling_on_sc=True, needs_layout_passes=True),
  )
  def gather_kernel(idx_hbm, table_hbm, out_hbm, idx_v, rows, gsem):
    wid = lax.axis_index("s") * NC + lax.axis_index("c")
    base = wid * B_PER_W
    maj0 = wid * (B_ROWS // NW)
    pltpu.sync_copy(idx_hbm.at[pl.ds(base, B_PER_W)], idx_v)

    def body(c, carry):
      pltpu.async_copy(
          table_hbm.at[idx_v.at[pl.ds(c * CHUNK, CHUNK)]], rows, gsem).wait()
      for m in range(CHUNK // B_COLS):
        pltpu.sync_copy(rows.at[pl.ds(m * B_COLS, B_COLS)],
                        out_hbm.at[maj0 + c * (CHUNK // B_COLS) + m])
      return carry

    lax.fori_loop(0, N_CHUNKS, body, 0)

  return gather_kernel


_GATHER = _make_kernel()


def kernel(x, table):
  idx = x.reshape(-1).astype(jnp.int32)
  return _GATHER(idx, table)
